# baseline (speedup 1.0000x reference)
import functools

import jax
import jax.numpy as jnp
from jax import lax
from jax.experimental import pallas as pl
from jax.experimental.pallas import tpu as pltpu
from jax.experimental.pallas import tpu_sc as plsc

TM = 512
TK = 1024
COMMIT = 0.25


def _rne_bf16(v):
    u = lax.bitcast_convert_type(v, jnp.uint32)
    r = (u + jnp.uint32(0x7FFF) + ((u >> 16) & jnp.uint32(1))) & jnp.uint32(0xFFFF0000)
    return lax.bitcast_convert_type(r, jnp.float32)


def _dist_argmin_body(z_ref, e_ref, z2_ref, e2_ref, idx_ref, loss_ref,
                      min_s, arg_s, *, nk):
    i = pl.program_id(0)
    k = pl.program_id(1)

    m = lax.dot_general(e_ref[...], z_ref[...].astype(jnp.float32),
                        (((1,), (0,)), ((), ())),
                        precision=lax.Precision.DEFAULT,
                        preferred_element_type=jnp.float32)
    d = (z2_ref[...].reshape(1, TM) - m) + e2_ref[...].reshape(TK, 1)

    local_min = jnp.min(d, axis=0)
    local_arg = jnp.argmin(d, axis=0).astype(jnp.int32) + k * TK

    @pl.when(k == 0)
    def _init():
        min_s[...] = local_min
        arg_s[...] = local_arg

    @pl.when(k == nk // 2)
    def _round_carry():
        min_s[...] = _rne_bf16(min_s[...])

    @pl.when(k != 0)
    def _update():
        better = local_min < min_s[...]
        min_s[...] = jnp.where(better, local_min, min_s[...])
        arg_s[...] = jnp.where(better, local_arg, arg_s[...])

    @pl.when((i == 0) & (k == 0))
    def _zero_loss():
        loss_ref[0, 0] = 0.0

    @pl.when(k == nk - 1)
    def _finish():
        idx_ref[...] = arg_s[...]
        loss_ref[0, 0] += jnp.sum(min_s[...])


def _dist_argmin(zbt, emb, z2, e2):
    d, n = zbt.shape
    kk = emb.shape[0]
    ni, nk = n // TM, kk // TK
    idx, loss_sum = pl.pallas_call(
        functools.partial(_dist_argmin_body, nk=nk),
        grid=(ni, nk),
        in_specs=[
            pl.BlockSpec((d, TM), lambda i, k: (0, i)),

            pl.BlockSpec((TK, d), lambda i, k: (k, 0)),
            pl.BlockSpec((1, TM), lambda i, k: (0, i)),
            pl.BlockSpec((TK, 1), lambda i, k: (k, 0)),
        ],
        out_specs=[
            pl.BlockSpec((TM,), lambda i, k: (i,)),
            pl.BlockSpec(memory_space=pltpu.SMEM, block_shape=(1, 1),
                         index_map=lambda i, k: (0, 0)),
        ],
        out_shape=[
            jax.ShapeDtypeStruct((n,), jnp.int32),
            jax.ShapeDtypeStruct((1, 1), jnp.float32),
        ],
        scratch_shapes=[
            pltpu.VMEM((TM,), jnp.float32),
            pltpu.VMEM((TM,), jnp.int32),
        ],
        compiler_params=pltpu.CompilerParams(
            dimension_semantics=("arbitrary", "arbitrary")),
    )(zbt, emb, z2.reshape(1, n), e2.reshape(kk, 1))
    return idx, loss_sum


def _make_sc_gather(d, b):
    info = plsc.get_sparse_core_info()
    nc, ns = info.num_cores, info.num_subcores
    nw = nc * ns
    b_per_w = b // nw
    mesh = plsc.VectorSubcoreMesh(core_axis_name="c", subcore_axis_name="s")

    @functools.partial(
        pl.kernel, mesh=mesh,
        out_type=jax.ShapeDtypeStruct((b, d), jnp.float32),
        scratch_types=[
            pltpu.VMEM((b_per_w,), jnp.int32),
            pltpu.VMEM((b_per_w, d), jnp.float32),
            pltpu.SemaphoreType.DMA,
        ],
    )
    def gather_k(table_hbm, idx_hbm, out_hbm, idx_v, rows_v, sem):
        wid = lax.axis_index("s") * nc + lax.axis_index("c")
        base = wid * b_per_w
        pltpu.sync_copy(idx_hbm.at[pl.ds(base, b_per_w)], idx_v)
        pltpu.async_copy(table_hbm.at[idx_v], rows_v, sem).wait()
        pltpu.sync_copy(rows_v, out_hbm.at[pl.ds(base, b_per_w)])

    return gather_k


def kernel(x, embedding_weight):
    b, d, h, w = x.shape
    n = b * h * w
    kk = embedding_weight.shape[0]

    zf = jnp.transpose(x, (0, 2, 3, 1)).reshape(n, d)
    z2 = jnp.sum(zf ** 2, axis=1, keepdims=True)
    e2 = jnp.sum(embedding_weight ** 2, axis=1)
    zbt = (2.0 * zf).astype(jnp.bfloat16).T

    idx, loss_sum = _dist_argmin(zbt, embedding_weight, z2, e2)

    gather_k = _make_sc_gather(d, n)
    quantized = gather_k(embedding_weight, idx)

    x_recon = jnp.transpose(quantized.reshape(b, h, w, d), (0, 3, 1, 2))
    vq_loss = (1.0 + COMMIT) * loss_sum[0, 0] / jnp.float32(n * d)
    return (x_recon, vq_loss)


# speedup vs baseline: 1.0050x; 1.0050x over previous
import functools

import jax
import jax.numpy as jnp
from jax import lax
from jax.experimental import pallas as pl
from jax.experimental.pallas import tpu as pltpu
from jax.experimental.pallas import tpu_sc as plsc

TM = 512
TK = 1024
COMMIT = 0.25


def _rne_bf16(v):
    u = lax.bitcast_convert_type(v, jnp.uint32)
    r = (u + jnp.uint32(0x7FFF) + ((u >> 16) & jnp.uint32(1))) & jnp.uint32(0xFFFF0000)
    return lax.bitcast_convert_type(r, jnp.float32)


def _dist_argmin_body(z_ref, e_ref, z2_ref, e2_ref, idx_ref, loss_ref,
                      min_s, arg_s, *, nk):
    i = pl.program_id(0)
    k = pl.program_id(1)

    m = lax.dot_general(e_ref[...], z_ref[0].astype(jnp.float32),
                        (((1,), (0,)), ((), ())),
                        precision=lax.Precision.DEFAULT,
                        preferred_element_type=jnp.float32)
    d = (z2_ref[...].reshape(1, TM) - m) + e2_ref[...].reshape(TK, 1)

    local_min = jnp.min(d, axis=0)
    local_arg = jnp.argmin(d, axis=0).astype(jnp.int32) + k * TK

    @pl.when(k == 0)
    def _init():
        min_s[...] = local_min
        arg_s[...] = local_arg

    @pl.when(k == nk // 2)
    def _round_carry():
        min_s[...] = _rne_bf16(min_s[...])

    @pl.when(k != 0)
    def _update():
        better = local_min < min_s[...]
        min_s[...] = jnp.where(better, local_min, min_s[...])
        arg_s[...] = jnp.where(better, local_arg, arg_s[...])

    @pl.when((i == 0) & (k == 0))
    def _zero_loss():
        loss_ref[0, 0] = 0.0

    @pl.when(k == nk - 1)
    def _finish():
        idx_ref[...] = arg_s[...]
        loss_ref[0, 0] += jnp.sum(min_s[...])


def _dist_argmin(zb3, emb, z2, e2):
    b, d, hw = zb3.shape
    n = b * hw
    kk = emb.shape[0]
    ni, nk = n // TM, kk // TK
    tpb = hw // TM
    idx, loss_sum = pl.pallas_call(
        functools.partial(_dist_argmin_body, nk=nk),
        grid=(ni, nk),
        in_specs=[
            pl.BlockSpec((1, d, TM), lambda i, k: (i // tpb, 0, i % tpb)),

            pl.BlockSpec((TK, d), lambda i, k: (k, 0)),
            pl.BlockSpec((1, TM), lambda i, k: (0, i)),
            pl.BlockSpec((TK, 1), lambda i, k: (k, 0)),
        ],
        out_specs=[
            pl.BlockSpec((TM,), lambda i, k: (i,)),
            pl.BlockSpec(memory_space=pltpu.SMEM, block_shape=(1, 1),
                         index_map=lambda i, k: (0, 0)),
        ],
        out_shape=[
            jax.ShapeDtypeStruct((n,), jnp.int32),
            jax.ShapeDtypeStruct((1, 1), jnp.float32),
        ],
        scratch_shapes=[
            pltpu.VMEM((TM,), jnp.float32),
            pltpu.VMEM((TM,), jnp.int32),
        ],
        compiler_params=pltpu.CompilerParams(
            dimension_semantics=("arbitrary", "arbitrary")),
    )(zb3, emb, z2.reshape(1, n), e2.reshape(kk, 1))
    return idx, loss_sum


def _make_sc_gather(d, b):
    info = plsc.get_sparse_core_info()
    nc, ns = info.num_cores, info.num_subcores
    nw = nc * ns
    b_per_w = b // nw
    mesh = plsc.VectorSubcoreMesh(core_axis_name="c", subcore_axis_name="s")

    @functools.partial(
        pl.kernel, mesh=mesh,
        out_type=jax.ShapeDtypeStruct((b, d), jnp.float32),
        scratch_types=[
            pltpu.VMEM((b_per_w,), jnp.int32),
            pltpu.VMEM((b_per_w, d), jnp.float32),
            pltpu.SemaphoreType.DMA,
        ],
    )
    def gather_k(table_hbm, idx_hbm, out_hbm, idx_v, rows_v, sem):
        wid = lax.axis_index("s") * nc + lax.axis_index("c")
        base = wid * b_per_w
        pltpu.sync_copy(idx_hbm.at[pl.ds(base, b_per_w)], idx_v)
        pltpu.async_copy(table_hbm.at[idx_v], rows_v, sem).wait()
        pltpu.sync_copy(rows_v, out_hbm.at[pl.ds(base, b_per_w)])

    return gather_k


def kernel(x, embedding_weight):
    b, d, h, w = x.shape
    n = b * h * w
    kk = embedding_weight.shape[0]

    z2 = jnp.sum(x ** 2, axis=1)
    e2 = jnp.sum(embedding_weight ** 2, axis=1)
    zb3 = (2.0 * x).astype(jnp.bfloat16).reshape(b, d, h * w)

    idx, loss_sum = _dist_argmin(zb3, embedding_weight, z2, e2)

    gather_k = _make_sc_gather(d, n)
    quantized = gather_k(embedding_weight, idx)

    x_recon = jnp.transpose(quantized.reshape(b, h, w, d), (0, 3, 1, 2))
    vq_loss = (1.0 + COMMIT) * loss_sum[0, 0] / jnp.float32(n * d)
    return (x_recon, vq_loss)


# speedup vs baseline: 1.1453x; 1.1396x over previous
import functools

import jax
import jax.numpy as jnp
from jax import lax
from jax.experimental import pallas as pl
from jax.experimental.pallas import tpu as pltpu
from jax.experimental.pallas import tpu_sc as plsc

TM = 512
TK = 1024
COMMIT = 0.25


def _rne_bf16(v):
    u = lax.bitcast_convert_type(v, jnp.uint32)
    r = (u + jnp.uint32(0x7FFF) + ((u >> 16) & jnp.uint32(1))) & jnp.uint32(0xFFFF0000)
    return lax.bitcast_convert_type(r, jnp.float32)


def _dist_argmin_body(z_ref, e_ref, z2_ref, e2_ref, idx_ref, loss_ref,
                      min_s, arg_s, *, nk):
    k = pl.program_id(0)
    i = pl.program_id(1)

    m = lax.dot_general(e_ref[...], z_ref[0].astype(jnp.float32),
                        (((1,), (0,)), ((), ())),
                        precision=lax.Precision.DEFAULT,
                        preferred_element_type=jnp.float32)
    d = (z2_ref[...].reshape(1, TM) - m) + e2_ref[...].reshape(TK, 1)

    local_min = jnp.min(d, axis=0)
    local_arg = jnp.argmin(d, axis=0).astype(jnp.int32) + k * TK

    @pl.when(k == 0)
    def _init():
        min_s[i] = local_min
        arg_s[i] = local_arg

    @pl.when(k == nk // 2)
    def _round_carry():
        min_s[i] = _rne_bf16(min_s[i])

    @pl.when(k != 0)
    def _update():
        better = local_min < min_s[i]
        min_s[i] = jnp.where(better, local_min, min_s[i])
        arg_s[i] = jnp.where(better, local_arg, arg_s[i])

    @pl.when((i == 0) & (k == 0))
    def _zero_loss():
        loss_ref[0, 0] = 0.0

    @pl.when(k == nk - 1)
    def _finish():
        idx_ref[...] = arg_s[i]
        loss_ref[0, 0] += jnp.sum(min_s[i])


def _dist_argmin(zb3, emb, z2, e2):
    b, d, hw = zb3.shape
    n = b * hw
    kk = emb.shape[0]
    ni, nk = n // TM, kk // TK
    tpb = hw // TM
    idx, loss_sum = pl.pallas_call(
        functools.partial(_dist_argmin_body, nk=nk),
        grid=(nk, ni),
        in_specs=[
            pl.BlockSpec((1, d, TM), lambda k, i: (i // tpb, 0, i % tpb)),

            pl.BlockSpec((TK, d), lambda k, i: (k, 0)),
            pl.BlockSpec((1, TM), lambda k, i: (0, i)),
            pl.BlockSpec((TK, 1), lambda k, i: (k, 0)),
        ],
        out_specs=[
            pl.BlockSpec((TM,), lambda k, i: (i,)),
            pl.BlockSpec(memory_space=pltpu.SMEM, block_shape=(1, 1),
                         index_map=lambda k, i: (0, 0)),
        ],
        out_shape=[
            jax.ShapeDtypeStruct((n,), jnp.int32),
            jax.ShapeDtypeStruct((1, 1), jnp.float32),
        ],
        scratch_shapes=[
            pltpu.VMEM((ni, TM), jnp.float32),
            pltpu.VMEM((ni, TM), jnp.int32),
        ],
        compiler_params=pltpu.CompilerParams(
            dimension_semantics=("arbitrary", "arbitrary")),
    )(zb3, emb, z2.reshape(1, n), e2.reshape(kk, 1))
    return idx, loss_sum


def _make_sc_gather(d, b):
    info = plsc.get_sparse_core_info()
    nc, ns = info.num_cores, info.num_subcores
    nw = nc * ns
    b_per_w = b // nw
    mesh = plsc.VectorSubcoreMesh(core_axis_name="c", subcore_axis_name="s")

    @functools.partial(
        pl.kernel, mesh=mesh,
        out_type=jax.ShapeDtypeStruct((b, d), jnp.float32),
        scratch_types=[
            pltpu.VMEM((b_per_w,), jnp.int32),
            pltpu.VMEM((b_per_w, d), jnp.float32),
            pltpu.SemaphoreType.DMA,
        ],
    )
    def gather_k(table_hbm, idx_hbm, out_hbm, idx_v, rows_v, sem):
        wid = lax.axis_index("s") * nc + lax.axis_index("c")
        base = wid * b_per_w
        pltpu.sync_copy(idx_hbm.at[pl.ds(base, b_per_w)], idx_v)
        pltpu.async_copy(table_hbm.at[idx_v], rows_v, sem).wait()
        pltpu.sync_copy(rows_v, out_hbm.at[pl.ds(base, b_per_w)])

    return gather_k


def kernel(x, embedding_weight):
    b, d, h, w = x.shape
    n = b * h * w
    kk = embedding_weight.shape[0]

    z2 = jnp.sum(x ** 2, axis=1)
    e2 = jnp.sum(embedding_weight ** 2, axis=1)
    zb3 = (2.0 * x).astype(jnp.bfloat16).reshape(b, d, h * w)

    idx, loss_sum = _dist_argmin(zb3, embedding_weight, z2, e2)

    gather_k = _make_sc_gather(d, n)
    quantized = gather_k(embedding_weight, idx)

    x_recon = jnp.transpose(quantized.reshape(b, h, w, d), (0, 3, 1, 2))
    vq_loss = (1.0 + COMMIT) * loss_sum[0, 0] / jnp.float32(n * d)
    return (x_recon, vq_loss)
